# baseline (speedup 1.0000x reference)
import functools

import jax
import jax.numpy as jnp
from jax import lax
from jax.experimental import pallas as pl
from jax.experimental.pallas import tpu as pltpu
from jax.experimental.pallas import tpu_sc as plsc

_INFO = plsc.get_sparse_core_info()
NC = _INFO.num_cores
NS = _INFO.num_subcores
NW = NC * NS
L = _INFO.num_lanes

CH = 128
R = 4
K = 2


@functools.partial(jax.jit, static_argnames=("n", "d"))
def _gather(idx, lutp, n, d):
    per_w = n // NW
    n_ch = per_w // CH
    n_groups = n_ch // R
    assert per_w % CH == 0 and n_ch % R == 0 and n_groups >= 3
    assert d % L == 0
    mesh = plsc.VectorSubcoreMesh(core_axis_name="c", subcore_axis_name="s")

    @functools.partial(
        pl.kernel,
        out_type=jax.ShapeDtypeStruct((n, d), jnp.float32),
        mesh=mesh,
        compiler_params=pltpu.CompilerParams(needs_layout_passes=False),
        scratch_types=(
            [
                pltpu.VMEM((n_ch, CH), jnp.int32),
                pltpu.VMEM((R, CH), jnp.int32),
                pltpu.VMEM((R, CH, 2 * d), jnp.float32),
                pltpu.VMEM((2, CH, d), jnp.float32),
            ]
            + [pltpu.SemaphoreType.DMA] * (2 * R)
        ),
    )
    def k(idx_hbm, table_hbm, out_hbm, idx_v, pidx_v, rows_v, ext_v, *sems):
        gsem = sems[:R]
        wsem = sems[R:]
        cid = lax.axis_index("c")
        sid = lax.axis_index("s")
        wid = sid * NC + cid
        base = wid * per_w
        pltpu.sync_copy(idx_hbm.at[wid], idx_v)

        lane = lax.iota(jnp.int32, L)

        def gather_start(j, s):
            for c in range(CH // L):
                pidx_v[s, pl.ds(c * L, L)] = idx_v[j, pl.ds(c * L, L)] >> 1
            pltpu.async_copy(table_hbm.at[pidx_v.at[s]], rows_v.at[s], gsem[s])

        def gather_wait(s):
            pltpu.make_async_copy(
                table_hbm.at[pidx_v.at[s]], rows_v.at[s], gsem[s]
            ).wait()

        def extract(j, s, t):
            rows2 = rows_v.at[s]
            ext2 = ext_v.at[t]
            for g in range(CH // L):
                rowid = lane + (g * L)
                bits = (idx_v[j, pl.ds(g * L, L)] & 1) * d

                def body(ci, carry):
                    for q in range(4):
                        c = ci * 4 + q
                        cvec = lax.broadcast(c, (L,))
                        val = plsc.load_gather(rows2, [rowid, bits + cvec])
                        plsc.store_scatter(ext2, [rowid, cvec], val)
                    return carry

                lax.fori_loop(0, d // 4, body, 0)

        def wb_start(j, s, t):
            pltpu.async_copy(
                ext_v.at[t], out_hbm.at[pl.ds(base + j * CH, CH)], wsem[s]
            )

        def wb_wait(s):
            pltpu.make_async_copy(
                ext_v.at[0], out_hbm.at[pl.ds(0, CH)], wsem[s]
            ).wait()

        for s in range(K):
            gather_start(s, s)

        for u in range(R):
            jn = u + K
            if jn >= R:
                wb_wait(jn % R)
            gather_start(jn, jn % R)
            gather_wait(u)
            extract(u, u, u % 2)
            wb_start(u, u, u % 2)

        def group(g, carry):
            i0 = g * R
            for u in range(R):
                sn = (u + K) % R
                wb_wait(sn)
                gather_start(i0 + u + K, sn)
                gather_wait(u)
                extract(i0 + u, u, u % 2)
                wb_start(i0 + u, u, u % 2)
            return carry

        lax.fori_loop(1, n_groups - 1, group, 0)

        i0 = (n_groups - 1) * R
        for u in range(R):
            jn = i0 + u + K
            wb_wait((u + K) % R)
            if jn < n_ch:
                gather_start(jn, (u + K) % R)
            gather_wait(u)
            extract(i0 + u, u, u % 2)
            wb_start(i0 + u, u, u % 2)

        for u in range(K, R):
            wb_wait(u)

    return k(idx, lutp)


def kernel(x, lut):
    b, h = x.shape
    v, d = lut.shape
    n = b * h
    assert n % (NW * CH) == 0
    idx = x.reshape(NW, n // NW // CH, CH)
    lutp = lut.reshape(v // 2, 2 * d)
    out = _gather(idx, lutp, n, d)
    return out.reshape(b, h, d)


# speedup vs baseline: 1.0035x; 1.0035x over previous
import functools

import jax
import jax.numpy as jnp
from jax import lax
from jax.experimental import pallas as pl
from jax.experimental.pallas import tpu as pltpu
from jax.experimental.pallas import tpu_sc as plsc

_INFO = plsc.get_sparse_core_info()
NC = _INFO.num_cores
NS = _INFO.num_subcores
NW = NC * NS
L = _INFO.num_lanes

CH = 128
R = 4
K = 2


@functools.partial(jax.jit, static_argnames=("n", "d"))
def _gather(idx, lutp, n, d):
    per_w = n // NW
    n_ch = per_w // CH
    n_groups = n_ch // R
    assert per_w % CH == 0 and n_ch % R == 0 and n_groups >= 3
    assert d % L == 0
    mesh = plsc.VectorSubcoreMesh(core_axis_name="c", subcore_axis_name="s")

    @functools.partial(
        pl.kernel,
        out_type=jax.ShapeDtypeStruct((n, d), jnp.float32),
        mesh=mesh,
        compiler_params=pltpu.CompilerParams(needs_layout_passes=False),
        scratch_types=(
            [
                pltpu.VMEM((n_ch, CH), jnp.int32),
                pltpu.VMEM((R, CH), jnp.int32),
                pltpu.VMEM((R, CH, 2 * d), jnp.float32),
                pltpu.VMEM((2, CH, d), jnp.float32),
            ]
            + [pltpu.SemaphoreType.DMA] * (2 * R)
        ),
    )
    def k(idx_hbm, table_hbm, out_hbm, idx_v, pidx_v, rows_v, ext_v, *sems):
        gsem = sems[:R]
        wsem = sems[R:]
        cid = lax.axis_index("c")
        sid = lax.axis_index("s")
        wid = sid * NC + cid
        base = wid * per_w
        pltpu.sync_copy(idx_hbm.at[wid], idx_v)

        lane = lax.iota(jnp.int32, L)

        def gather_start(j, s):
            for c in range(CH // L):
                pidx_v[s, pl.ds(c * L, L)] = idx_v[j, pl.ds(c * L, L)] >> 1
            pltpu.async_copy(table_hbm.at[pidx_v.at[s]], rows_v.at[s], gsem[s])

        def gather_wait(s):
            pltpu.make_async_copy(
                table_hbm.at[pidx_v.at[s]], rows_v.at[s], gsem[s]
            ).wait()

        def extract(j, s, t):
            rows2 = rows_v.at[s]
            ext2 = ext_v.at[t]

            def grp(g, carry):
                g16 = g * L
                rowid = lane + g16
                bits = (idx_v[j, pl.ds(g16, L)] & 1) * d

                def body(ci, c2):
                    cv0 = lax.broadcast(ci * L, (L,))
                    for q in range(L):
                        cvec = cv0 + q
                        val = plsc.load_gather(rows2, [rowid, bits + cvec])
                        plsc.store_scatter(ext2, [rowid, cvec], val)
                    return c2

                lax.fori_loop(0, d // L, body, 0)
                return carry

            lax.fori_loop(0, CH // L, grp, 0)

        def wb_start(j, s, t):
            pltpu.async_copy(
                ext_v.at[t], out_hbm.at[pl.ds(base + j * CH, CH)], wsem[s]
            )

        def wb_wait(s):
            pltpu.make_async_copy(
                ext_v.at[0], out_hbm.at[pl.ds(0, CH)], wsem[s]
            ).wait()

        for s in range(K):
            gather_start(s, s)

        for u in range(R):
            jn = u + K
            if jn >= R:
                wb_wait(jn % R)
            gather_start(jn, jn % R)
            gather_wait(u)
            extract(u, u, u % 2)
            wb_start(u, u, u % 2)

        def group(g, carry):
            i0 = g * R
            for u in range(R):
                sn = (u + K) % R
                wb_wait(sn)
                gather_start(i0 + u + K, sn)
                gather_wait(u)
                extract(i0 + u, u, u % 2)
                wb_start(i0 + u, u, u % 2)
            return carry

        lax.fori_loop(1, n_groups - 1, group, 0)

        i0 = (n_groups - 1) * R
        for u in range(R):
            jn = i0 + u + K
            wb_wait((u + K) % R)
            if jn < n_ch:
                gather_start(jn, (u + K) % R)
            gather_wait(u)
            extract(i0 + u, u, u % 2)
            wb_start(i0 + u, u, u % 2)

        for u in range(K, R):
            wb_wait(u)

    return k(idx, lutp)


def kernel(x, lut):
    b, h = x.shape
    v, d = lut.shape
    n = b * h
    assert n % (NW * CH) == 0
    idx = x.reshape(NW, n // NW // CH, CH)
    lutp = lut.reshape(v // 2, 2 * d)
    out = _gather(idx, lutp, n, d)
    return out.reshape(b, h, d)
